# baseline (speedup 1.0000x reference)
import functools

import jax
import jax.numpy as jnp
from jax import lax
from jax.experimental import pallas as pl
from jax.experimental.pallas import tpu as pltpu
from jax.experimental.pallas import tpu_sc as plsc

_ROWS = 128
_N = 8192
_TOPK1 = 65
_EPS = 1e-07
_NEG_INF = float("-inf")

_NC = 2
_NS = 16
_NW = _NC * _NS
_RPW = _ROWS // _NW
_NVEC = _N // 16


def _iota16():
    return lax.broadcasted_iota(jnp.int32, (16,), 0)


def _splat_from_sorted(s, j):
    return jnp.max(jnp.where(_iota16() == j, s, jnp.float32(_NEG_INF)))


def _partition(src, src_base, n, nvec, pivot, cand, dst_a, dst_b):
    pivot_v = jnp.broadcast_to(pivot, (16,))
    iota = _iota16()
    zeros = jnp.zeros((16,), jnp.int32)

    def body(i, carry):
        offa, offb = carry
        x = src[pl.ds(src_base + i * 16, 16)]
        valid = (i * 16 + iota) < n
        ma = jnp.logical_and(x > pivot_v, valid)
        mb = jnp.logical_and(x < pivot_v, valid)
        ia = ma.astype(jnp.int32)
        ib = mb.astype(jnp.int32)
        csa = plsc.cumsum(ia)
        csb = plsc.cumsum(ib)
        pa = dst_a + offa + csa - ia
        pb = dst_b + offb + csb - ib
        plsc.store_scatter(cand, [pa], x, mask=ma)
        plsc.store_scatter(cand, [pb], x, mask=mb)
        offa = offa + plsc.all_reduce_population_count(ma)
        offb = offb + plsc.all_reduce_population_count(mb)
        return offa, offb

    offa, offb = lax.fori_loop(0, nvec, body, (zeros, zeros))
    return jnp.max(offa), jnp.max(offb)


def _choose_pivot_j(n, rank):
    rank_bot = n + 1 - rank
    m = n // 8 + 8
    np1 = n + 1
    t_top = jnp.minimum(rank + m, n - 1)
    j_top = 16 - jnp.clip((17 * t_top + np1 // 2) // np1, 1, 16)
    t_bot = jnp.minimum(rank_bot + m, n - 1)
    j_bot = jnp.clip((17 * t_bot + np1 // 2) // np1, 1, 16) - 1
    return jnp.where(rank <= rank_bot, j_top, j_bot)


def _advance(n, rank, c, nb, pivot, dst_a, dst_b, done, delta):
    e = n - c - nb
    go_above = rank <= c
    now_done = jnp.logical_and(jnp.logical_not(go_above), rank <= c + e)
    now_done = jnp.logical_and(now_done, done == 0)
    delta = jnp.where(now_done, pivot, delta)
    n2 = jnp.where(go_above, c, nb)
    rank2 = jnp.where(go_above, rank, rank - (c + e))
    src2 = jnp.where(go_above, dst_a, dst_b)
    done2 = jnp.where(now_done, jnp.int32(1), done)
    return n2, rank2, src2, done2, delta


def _select_threshold(xbuf, row_base, cand):
    iota = _iota16()

    s0 = lax.sort(plsc.load_gather(xbuf, [row_base + iota * (_N // 16)]))
    j0 = _choose_pivot_j(jnp.int32(_N), jnp.int32(_TOPK1))
    piv0 = _splat_from_sorted(s0, j0)
    c0, nb0 = _partition(
        xbuf, row_base, jnp.int32(_N), _NVEC, piv0, cand, 0, _N
    )
    n, rank, src_base, done, delta = _advance(
        jnp.int32(_N), jnp.int32(_TOPK1), c0, nb0, piv0,
        jnp.int32(0), jnp.int32(_N), jnp.int32(0), jnp.float32(0.0)
    )

    def cond(st):
        n, rank, src_base, phase, done, delta = st
        return jnp.logical_and(done == 0, n > 16)

    def body(st):
        n, rank, src_base, phase, done, delta = st
        s = lax.sort(
            plsc.load_gather(cand, [src_base + _iota16() * (n // 16)])
        )
        j = _choose_pivot_j(n, rank)
        pivot = _splat_from_sorted(s, j)
        dst_a = (1 - phase) * (2 * _N)
        dst_b = dst_a + _N
        nvec = (n + 15) // 16
        c, nb = _partition(cand, src_base, n, nvec, pivot, cand, dst_a, dst_b)
        n2, rank2, src2, done2, delta2 = _advance(
            n, rank, c, nb, pivot, dst_a, dst_b, done, delta
        )
        return n2, rank2, src2, 1 - phase, done2, delta2

    st = (n, rank, src_base, jnp.int32(0), done, delta)
    n, rank, src_base, phase, done, delta = lax.while_loop(cond, body, st)

    v = cand[pl.ds(src_base, 16)]
    v = jnp.where(iota < n, v, jnp.float32(_NEG_INF))
    sv = lax.sort(v)
    dsmall = _splat_from_sorted(sv, 16 - rank)
    return jnp.where(done == 0, dsmall, delta)


def _normalize_row(xbuf, row_base, delta):
    delta_v = jnp.broadcast_to(delta, (16,))
    zero = jnp.zeros((16,), jnp.float32)

    def body_a(i, acc):
        sl = pl.ds(row_base + i * 16, 16)
        w = jnp.maximum(xbuf[sl] - delta_v, 0.0)
        xbuf[sl] = w
        return acc + w

    acc = lax.fori_loop(0, _NVEC, body_a, zero)
    denom_v = jnp.broadcast_to(jnp.sum(acc) + jnp.float32(_EPS), (16,))
    scale_v = jnp.ones((16,), jnp.float32) / denom_v

    def body_b(i, carry):
        sl = pl.ds(row_base + i * 16, 16)
        xbuf[sl] = xbuf[sl] * scale_v
        return carry

    lax.fori_loop(0, _NVEC, body_b, jnp.int32(0))


def _sc_body(x_hbm, out_hbm, xbuf, cand):
    wid = lax.axis_index("s") * _NC + lax.axis_index("c")
    base = wid * (_RPW * _N)
    pltpu.sync_copy(x_hbm.at[pl.ds(base, _RPW * _N)], xbuf)
    for r in range(_RPW):
        row_base = r * _N
        delta = _select_threshold(xbuf, row_base, cand)
        _normalize_row(xbuf, row_base, delta)
    pltpu.sync_copy(xbuf, out_hbm.at[pl.ds(base, _RPW * _N)])


@jax.jit
def kernel(attn_s):
    b, one, n = attn_s.shape
    x = attn_s.reshape(b * n)
    mesh = plsc.VectorSubcoreMesh(
        core_axis_name="c", subcore_axis_name="s",
        num_cores=_NC, num_subcores=_NS,
    )
    out = pl.kernel(
        _sc_body,
        out_type=jax.ShapeDtypeStruct((b * n,), jnp.float32),
        mesh=mesh,
        scratch_types=[
            pltpu.VMEM((_RPW * _N,), jnp.float32),
            pltpu.VMEM((4 * _N,), jnp.float32),
        ],
        compiler_params=pltpu.CompilerParams(needs_layout_passes=False),
    )(x)
    return out.reshape(b, one, n)


# speedup vs baseline: 2.0577x; 2.0577x over previous
import functools

import jax
import jax.numpy as jnp
from jax import lax
from jax.experimental import pallas as pl
from jax.experimental.pallas import tpu as pltpu
from jax.experimental.pallas import tpu_sc as plsc

_ROWS = 128
_N = 8192
_TOPK1 = 65
_EPS = 1e-07
_NEG_INF = float("-inf")

_NC = 2
_NS = 16
_NW = _NC * _NS
_RPW = _ROWS // _NW
_NVEC = _N // 16


def _iota16():
    return lax.broadcasted_iota(jnp.int32, (16,), 0)


def _lane(s, j):
    return jnp.max(jnp.where(_iota16() == j, s, jnp.float32(_NEG_INF)))


def _pivot_j_static(n, rank):
    rank_bot = n + 1 - rank
    m = n // 8 + 8
    if rank <= rank_bot:
        t = min(rank + m, n - 1)
        return 16 - min(max((17 * t + (n + 1) // 2) // (n + 1), 1), 16)
    t = min(rank_bot + m, n - 1)
    return min(max((17 * t + (n + 1) // 2) // (n + 1), 1), 16) - 1


def _pivot_j(n, rank):
    rank_bot = n + 1 - rank
    m = n // 8 + 8
    np1 = n + 1
    t_top = jnp.minimum(rank + m, n - 1)
    j_top = 16 - jnp.clip((17 * t_top + np1 // 2) // np1, 1, 16)
    t_bot = jnp.minimum(rank_bot + m, n - 1)
    j_bot = jnp.clip((17 * t_bot + np1 // 2) // np1, 1, 16) - 1
    return jnp.where(rank <= rank_bot, j_top, j_bot)


def _partition_full(src, src_base, n, pivot, cand, dst_a, dst_b):
    pivot_v = jnp.broadcast_to(pivot, (16,))
    iota = _iota16()
    zi = jnp.zeros((16,), jnp.int32)
    zf = jnp.zeros((16,), jnp.float32)

    def body(i, carry):
        offa, offb, sacc = carry
        x = src[pl.ds(src_base + i * 16, 16)]
        valid = (i * 16 + iota) < n
        ma = jnp.logical_and(x > pivot_v, valid)
        mb = jnp.logical_and(x < pivot_v, valid)
        ia = ma.astype(jnp.int32)
        ib = mb.astype(jnp.int32)
        csa = plsc.cumsum(ia)
        csb = plsc.cumsum(ib)
        plsc.store_scatter(cand, [dst_a + offa + csa - ia], x, mask=ma)
        plsc.store_scatter(cand, [dst_b + offb + csb - ib], x, mask=mb)
        offa = offa + plsc.all_reduce_population_count(ma)
        offb = offb + plsc.all_reduce_population_count(mb)
        sacc = sacc + jnp.where(ma, x, 0.0)
        return offa, offb, sacc

    nvec = (n + 15) // 16
    offa, offb, sacc = lax.fori_loop(0, nvec, body, (zi, zi, zf))
    return jnp.max(offa), jnp.max(offb), jnp.sum(sacc)


def _select_threshold(xbuf, row_base, cand):
    iota = _iota16()

    s0 = lax.sort(plsc.load_gather(xbuf, [row_base + iota * (_N // 16)]))
    piv0 = _lane(s0, _pivot_j_static(_N, _TOPK1))
    piv0_v = jnp.broadcast_to(piv0, (16,))
    zi = jnp.zeros((16,), jnp.int32)

    @plsc.parallel_loop(0, _N, 16, unroll=8, carry=zi)
    def _p1(i, offa):
        x = xbuf[pl.ds(row_base + i, 16)]
        ma = x > piv0_v
        ia = ma.astype(jnp.int32)
        csa = plsc.cumsum(ia)
        plsc.store_scatter(cand, [offa + csa - ia], x, mask=ma)
        return offa + plsc.all_reduce_population_count(ma)

    c0 = jnp.max(_p1)

    def _common(_):
        return (c0, jnp.int32(_TOPK1), jnp.int32(0), jnp.int32(0),
                jnp.float32(0.0), jnp.float32(0.0), jnp.int32(0))

    def _rare(_):
        c, nb, sa = _partition_full(
            xbuf, row_base, jnp.int32(_N), piv0, cand, 0, _N
        )
        e = _N - c - nb
        now_done = _TOPK1 <= c + e
        delta = jnp.where(now_done, piv0, jnp.float32(0.0))
        ef = e.astype(jnp.float32)
        s_acc = jnp.where(now_done, sa, sa + piv0 * ef)
        c_acc = jnp.where(now_done, c, c + e)
        return (nb, jnp.int32(_TOPK1) - (c + e), jnp.int32(_N),
                jnp.where(now_done, jnp.int32(1), jnp.int32(0)),
                delta, s_acc, c_acc)

    n, rank, src_base, done, delta, s_acc, c_acc = lax.cond(
        c0 >= _TOPK1, _common, _rare, jnp.int32(0)
    )

    def cond(st):
        n, rank, src_base, phase, done, delta, s_acc, c_acc = st
        return jnp.logical_and(done == 0, n > 16)

    def body(st):
        n, rank, src_base, phase, done, delta, s_acc, c_acc = st
        s = lax.sort(
            plsc.load_gather(cand, [src_base + _iota16() * (n // 16)])
        )
        pivot = _lane(s, _pivot_j(n, rank))
        dst_a = (1 - phase) * (2 * _N)
        dst_b = dst_a + _N
        c, nb, sa = _partition_full(cand, src_base, n, pivot, cand,
                                    dst_a, dst_b)
        e = n - c - nb
        go_above = rank <= c
        now_done = jnp.logical_and(jnp.logical_not(go_above), rank <= c + e)
        delta = jnp.where(now_done, pivot, delta)
        ef = e.astype(jnp.float32)
        ds = jnp.where(go_above, jnp.float32(0.0),
                       jnp.where(now_done, sa, sa + pivot * ef))
        dc = jnp.where(go_above, jnp.int32(0),
                       jnp.where(now_done, c, c + e))
        n2 = jnp.where(go_above, c, nb)
        rank2 = jnp.where(go_above, rank, rank - (c + e))
        src2 = jnp.where(go_above, dst_a, dst_b)
        done2 = jnp.where(now_done, jnp.int32(1), done)
        return (n2, rank2, src2, 1 - phase, done2, delta,
                s_acc + ds, c_acc + dc)

    st = (n, rank, src_base, jnp.int32(0), done, delta, s_acc, c_acc)
    st = lax.while_loop(cond, body, st)
    n, rank, src_base, phase, done, delta, s_acc, c_acc = st

    v = cand[pl.ds(src_base, 16)]
    v = jnp.where(iota < n, v, jnp.float32(_NEG_INF))
    sv = lax.sort(v)
    dsmall = _lane(sv, 16 - rank)
    delta = jnp.where(done == 0, dsmall, delta)
    mv = jnp.logical_and(done == 0, v > jnp.broadcast_to(delta, (16,)))
    s_acc = s_acc + jnp.sum(jnp.where(mv, v, jnp.float32(0.0)))
    c_acc = c_acc + jnp.max(plsc.all_reduce_population_count(mv))
    return delta, s_acc, c_acc


def _normalize_row(xbuf, row_base, delta, s_acc, c_acc):
    wsum = s_acc - delta * c_acc.astype(jnp.float32)
    denom_v = jnp.broadcast_to(wsum + jnp.float32(_EPS), (16,))
    scale_v = jnp.ones((16,), jnp.float32) / denom_v
    delta_v = jnp.broadcast_to(delta, (16,))

    @plsc.parallel_loop(0, _N, 16, unroll=8)
    def _p2(i):
        sl = pl.ds(row_base + i, 16)
        xbuf[sl] = jnp.maximum(xbuf[sl] - delta_v, 0.0) * scale_v


def _sc_body(x_hbm, out_hbm, xbuf, cand):
    wid = lax.axis_index("s") * _NC + lax.axis_index("c")
    base = wid * (_RPW * _N)
    pltpu.sync_copy(x_hbm.at[pl.ds(base, _RPW * _N)], xbuf)
    for r in range(_RPW):
        row_base = r * _N
        delta, s_acc, c_acc = _select_threshold(xbuf, row_base, cand)
        _normalize_row(xbuf, row_base, delta, s_acc, c_acc)
    pltpu.sync_copy(xbuf, out_hbm.at[pl.ds(base, _RPW * _N)])


@jax.jit
def kernel(attn_s):
    b, one, n = attn_s.shape
    x = attn_s.reshape(b * n)
    mesh = plsc.VectorSubcoreMesh(
        core_axis_name="c", subcore_axis_name="s",
        num_cores=_NC, num_subcores=_NS,
    )
    out = pl.kernel(
        _sc_body,
        out_type=jax.ShapeDtypeStruct((b * n,), jnp.float32),
        mesh=mesh,
        scratch_types=[
            pltpu.VMEM((_RPW * _N,), jnp.float32),
            pltpu.VMEM((4 * _N,), jnp.float32),
        ],
        compiler_params=pltpu.CompilerParams(needs_layout_passes=False),
    )(x)
    return out.reshape(b, one, n)


# speedup vs baseline: 2.0600x; 1.0011x over previous
import functools

import jax
import jax.numpy as jnp
from jax import lax
from jax.experimental import pallas as pl
from jax.experimental.pallas import tpu as pltpu
from jax.experimental.pallas import tpu_sc as plsc

_ROWS = 128
_N = 8192
_TOPK1 = 65
_EPS = 1e-07
_NEG_INF = float("-inf")

_NC = 2
_NS = 16
_NW = _NC * _NS
_RPW = _ROWS // _NW
_NVEC = _N // 16
_CB = _RPW * _N


def _iota16():
    return lax.broadcasted_iota(jnp.int32, (16,), 0)


def _lane(s, j):
    return jnp.max(jnp.where(_iota16() == j, s, jnp.float32(_NEG_INF)))


def _pivot_j_static(n, rank):
    rank_bot = n + 1 - rank
    m = n // 8 + 8
    if rank <= rank_bot:
        t = min(rank + m, n - 1)
        return 16 - min(max((17 * t + (n + 1) // 2) // (n + 1), 1), 16)
    t = min(rank_bot + m, n - 1)
    return min(max((17 * t + (n + 1) // 2) // (n + 1), 1), 16) - 1


def _pivot_j(n, rank):
    rank_bot = n + 1 - rank
    m = n // 8 + 8
    np1 = n + 1
    t_top = jnp.minimum(rank + m, n - 1)
    j_top = 16 - jnp.clip((17 * t_top + np1 // 2) // np1, 1, 16)
    t_bot = jnp.minimum(rank_bot + m, n - 1)
    j_bot = jnp.clip((17 * t_bot + np1 // 2) // np1, 1, 16) - 1
    return jnp.where(rank <= rank_bot, j_top, j_bot)


def _select_threshold(buf, row_base):
    iota = _iota16()

    s0 = lax.sort(plsc.load_gather(buf, [row_base + iota * (_N // 16)]))
    piv0 = _lane(s0, _pivot_j_static(_N, _TOPK1))
    piv0_v = jnp.broadcast_to(piv0, (16,))
    zi = jnp.zeros((16,), jnp.int32)

    @plsc.parallel_loop(0, _N, 16, unroll=8, carry=zi)
    def _p1(i, offa):
        x = buf[pl.ds(row_base + i, 16)]
        ma = x > piv0_v
        ia = ma.astype(jnp.int32)
        csa = plsc.cumsum(ia)
        plsc.store_scatter(buf, [_CB + offa + csa - ia], x, mask=ma)
        return offa + plsc.all_reduce_population_count(ma)

    c0 = jnp.max(_p1)

    ok = c0 >= _TOPK1
    n = jnp.where(ok, c0, jnp.int32(_N))
    src_base = jnp.where(ok, jnp.int32(_CB), jnp.int32(row_base))

    def cond(st):
        n, rank, src_base, phase, done, delta, s_acc, c_acc = st
        return jnp.logical_and(done == 0, n > 16)

    def body(st):
        n, rank, src_base, phase, done, delta, s_acc, c_acc = st
        smp = lax.sort(
            plsc.load_gather(buf, [src_base + _iota16() * (n // 16)])
        )
        pivot = _lane(smp, _pivot_j(n, rank))
        pivot_v = jnp.broadcast_to(pivot, (16,))
        dst_a = _CB + _N + phase * (2 * _N)
        dst_b = jnp.where(phase == 0, dst_a + _N, jnp.int32(_CB))
        zi = jnp.zeros((16,), jnp.int32)
        zf = jnp.zeros((16,), jnp.float32)

        def pbody(i, carry):
            offa, offb, sacc = carry
            x = buf[pl.ds(src_base + i * 16, 16)]
            valid = (i * 16 + iota) < n
            ma = jnp.logical_and(x > pivot_v, valid)
            mb = jnp.logical_and(x < pivot_v, valid)
            ia = ma.astype(jnp.int32)
            ib = mb.astype(jnp.int32)
            csa = plsc.cumsum(ia)
            csb = plsc.cumsum(ib)
            plsc.store_scatter(buf, [dst_a + offa + csa - ia], x, mask=ma)
            plsc.store_scatter(buf, [dst_b + offb + csb - ib], x, mask=mb)
            offa = offa + plsc.all_reduce_population_count(ma)
            offb = offb + plsc.all_reduce_population_count(mb)
            sacc = sacc + jnp.where(ma, x, 0.0)
            return offa, offb, sacc

        nvec = (n + 15) // 16
        offa, offb, sacc = lax.fori_loop(0, nvec, pbody, (zi, zi, zf))
        c = jnp.max(offa)
        nb = jnp.max(offb)
        sa = jnp.sum(sacc)

        e = n - c - nb
        go_above = rank <= c
        now_done = jnp.logical_and(jnp.logical_not(go_above), rank <= c + e)
        delta = jnp.where(now_done, pivot, delta)
        ef = e.astype(jnp.float32)
        ds = jnp.where(go_above, jnp.float32(0.0),
                       jnp.where(now_done, sa, sa + pivot * ef))
        dc = jnp.where(go_above, jnp.int32(0),
                       jnp.where(now_done, c, c + e))
        n2 = jnp.where(go_above, c, nb)
        rank2 = jnp.where(go_above, rank, rank - (c + e))
        src2 = jnp.where(go_above, dst_a, dst_b)
        done2 = jnp.where(now_done, jnp.int32(1), done)
        return (n2, rank2, src2, 1 - phase, done2, delta,
                s_acc + ds, c_acc + dc)

    st = (n, jnp.int32(_TOPK1), src_base, jnp.int32(0), jnp.int32(0),
          jnp.float32(0.0), jnp.float32(0.0), jnp.int32(0))
    st = lax.while_loop(cond, body, st)
    n, rank, src_base, phase, done, delta, s_acc, c_acc = st

    v = buf[pl.ds(src_base, 16)]
    v = jnp.where(iota < n, v, jnp.float32(_NEG_INF))
    sv = lax.sort(v)
    dsmall = _lane(sv, 16 - rank)
    delta = jnp.where(done == 0, dsmall, delta)
    mv = jnp.logical_and(done == 0, v > jnp.broadcast_to(delta, (16,)))
    s_acc = s_acc + jnp.sum(jnp.where(mv, v, jnp.float32(0.0)))
    c_acc = c_acc + jnp.max(plsc.all_reduce_population_count(mv))
    return delta, s_acc, c_acc


def _normalize_row(buf, row_base, delta, s_acc, c_acc):
    wsum = s_acc - delta * c_acc.astype(jnp.float32)
    denom_v = jnp.broadcast_to(wsum + jnp.float32(_EPS), (16,))
    scale_v = jnp.ones((16,), jnp.float32) / denom_v
    delta_v = jnp.broadcast_to(delta, (16,))

    @plsc.parallel_loop(0, _N, 16, unroll=8)
    def _p2(i):
        sl = pl.ds(row_base + i, 16)
        buf[sl] = jnp.maximum(buf[sl] - delta_v, 0.0) * scale_v


def _sc_body(x_hbm, out_hbm, buf):
    wid = lax.axis_index("s") * _NC + lax.axis_index("c")
    base = wid * (_RPW * _N)
    pltpu.sync_copy(x_hbm.at[pl.ds(base, _RPW * _N)], buf.at[pl.ds(0, _CB)])
    for r in range(_RPW):
        row_base = r * _N
        delta, s_acc, c_acc = _select_threshold(buf, row_base)
        _normalize_row(buf, row_base, delta, s_acc, c_acc)
    pltpu.sync_copy(buf.at[pl.ds(0, _CB)], out_hbm.at[pl.ds(base, _RPW * _N)])


@jax.jit
def kernel(attn_s):
    b, one, n = attn_s.shape
    x = attn_s.reshape(b * n)
    mesh = plsc.VectorSubcoreMesh(
        core_axis_name="c", subcore_axis_name="s",
        num_cores=_NC, num_subcores=_NS,
    )
    out = pl.kernel(
        _sc_body,
        out_type=jax.ShapeDtypeStruct((b * n,), jnp.float32),
        mesh=mesh,
        scratch_types=[
            pltpu.VMEM((_CB + 4 * _N,), jnp.float32),
        ],
        compiler_params=pltpu.CompilerParams(needs_layout_passes=False),
    )(x)
    return out.reshape(b, one, n)


# speedup vs baseline: 2.4067x; 1.1683x over previous
import functools

import jax
import jax.numpy as jnp
from jax import lax
from jax.experimental import pallas as pl
from jax.experimental.pallas import tpu as pltpu
from jax.experimental.pallas import tpu_sc as plsc

_ROWS = 128
_N = 8192
_TOPK1 = 65
_EPS = 1e-07
_NEG_INF = float("-inf")

_NC = 2
_NS = 16
_NW = _NC * _NS
_RPW = _ROWS // _NW
_NVEC = _N // 16
_CB = _RPW * _N


def _iota16():
    return lax.broadcasted_iota(jnp.int32, (16,), 0)


def _lane(s, j):
    return jnp.max(jnp.where(_iota16() == j, s, jnp.float32(_NEG_INF)))


def _pivot_j_static(n, rank):
    rank_bot = n + 1 - rank
    m = n // 8 + 8
    if rank <= rank_bot:
        t = min(rank + m, n - 1)
        return 16 - min(max((17 * t + (n + 1) // 2) // (n + 1), 1), 16)
    t = min(rank_bot + m, n - 1)
    return min(max((17 * t + (n + 1) // 2) // (n + 1), 1), 16) - 1


def _pivot_j(n, rank):
    rank_bot = n + 1 - rank
    m = n // 8 + 8
    np1 = n + 1
    t_top = jnp.minimum(rank + m, n - 1)
    j_top = 16 - jnp.clip((17 * t_top + np1 // 2) // np1, 1, 16)
    t_bot = jnp.minimum(rank_bot + m, n - 1)
    j_bot = jnp.clip((17 * t_bot + np1 // 2) // np1, 1, 16) - 1
    return jnp.where(rank <= rank_bot, j_top, j_bot)


def _merge16(a, b):
    rb = lax.rev(b, (0,))
    return lax.sort(jnp.minimum(a, rb)), lax.sort(jnp.maximum(a, rb))


def _merge32(alo, ahi, blo, bhi):
    rbl = lax.rev(bhi, (0,))
    rbh = lax.rev(blo, (0,))
    l0 = jnp.minimum(alo, rbl)
    l1 = jnp.minimum(ahi, rbh)
    h0 = jnp.maximum(alo, rbl)
    h1 = jnp.maximum(ahi, rbh)
    w0 = lax.sort(jnp.minimum(l0, l1))
    w1 = lax.sort(jnp.maximum(l0, l1))
    w2 = lax.sort(jnp.minimum(h0, h1))
    w3 = lax.sort(jnp.maximum(h0, h1))
    return w0, w1, w2, w3


def _select_threshold(buf, row_base):
    iota = _iota16()

    mx = plsc.load_gather(buf, [row_base + iota * (_N // 128)])
    for v in range(1, 8):
        mx = jnp.maximum(
            mx,
            plsc.load_gather(
                buf, [row_base + v * (_N // 8) + iota * (_N // 128)]
            ),
        )
    piv0 = _lane(lax.sort(mx), 11)
    piv0_v = jnp.broadcast_to(piv0, (16,))
    zi = jnp.zeros((16,), jnp.int32)

    @plsc.parallel_loop(0, _N, 16, unroll=8, carry=zi)
    def _p1(i, offa):
        x = buf[pl.ds(row_base + i, 16)]
        ma = x > piv0_v
        ia = ma.astype(jnp.int32)
        csa = plsc.cumsum(ia)
        plsc.store_scatter(buf, [_CB + offa + csa - ia], x, mask=ma)
        return offa + plsc.all_reduce_population_count(ma)

    c0 = jnp.max(_p1)

    ok = c0 >= _TOPK1
    n = jnp.where(ok, c0, jnp.int32(_N))
    src_base = jnp.where(ok, jnp.int32(_CB), jnp.int32(row_base))

    def cond(st):
        n, rank, src_base, phase, done, delta, s_acc, c_acc = st
        return jnp.logical_and(done == 0, n > 64)

    def body(st):
        n, rank, src_base, phase, done, delta, s_acc, c_acc = st
        smp = lax.sort(
            plsc.load_gather(buf, [src_base + _iota16() * (n // 16)])
        )
        pivot = _lane(smp, _pivot_j(n, rank))
        pivot_v = jnp.broadcast_to(pivot, (16,))
        dst_a = _CB + _N + phase * (2 * _N)
        dst_b = jnp.where(phase == 0, dst_a + _N, jnp.int32(_CB))
        zi = jnp.zeros((16,), jnp.int32)
        zf = jnp.zeros((16,), jnp.float32)

        def pbody(i, carry):
            offa, offb, sacc = carry
            x = buf[pl.ds(src_base + i * 16, 16)]
            valid = (i * 16 + iota) < n
            ma = jnp.logical_and(x > pivot_v, valid)
            mb = jnp.logical_and(x < pivot_v, valid)
            ia = ma.astype(jnp.int32)
            ib = mb.astype(jnp.int32)
            csa = plsc.cumsum(ia)
            csb = plsc.cumsum(ib)
            plsc.store_scatter(buf, [dst_a + offa + csa - ia], x, mask=ma)
            plsc.store_scatter(buf, [dst_b + offb + csb - ib], x, mask=mb)
            offa = offa + plsc.all_reduce_population_count(ma)
            offb = offb + plsc.all_reduce_population_count(mb)
            sacc = sacc + jnp.where(ma, x, 0.0)
            return offa, offb, sacc

        nvec = (n + 15) // 16
        offa, offb, sacc = lax.fori_loop(0, nvec, pbody, (zi, zi, zf))
        c = jnp.max(offa)
        nb = jnp.max(offb)
        sa = jnp.sum(sacc)

        e = n - c - nb
        go_above = rank <= c
        now_done = jnp.logical_and(jnp.logical_not(go_above), rank <= c + e)
        delta = jnp.where(now_done, pivot, delta)
        ef = e.astype(jnp.float32)
        ds = jnp.where(go_above, jnp.float32(0.0),
                       jnp.where(now_done, sa, sa + pivot * ef))
        dc = jnp.where(go_above, jnp.int32(0),
                       jnp.where(now_done, c, c + e))
        n2 = jnp.where(go_above, c, nb)
        rank2 = jnp.where(go_above, rank, rank - (c + e))
        src2 = jnp.where(go_above, dst_a, dst_b)
        done2 = jnp.where(now_done, jnp.int32(1), done)
        return (n2, rank2, src2, 1 - phase, done2, delta,
                s_acc + ds, c_acc + dc)

    st = (n, jnp.int32(_TOPK1), src_base, jnp.int32(0), jnp.int32(0),
          jnp.float32(0.0), jnp.float32(0.0), jnp.int32(0))
    st = lax.while_loop(cond, body, st)
    n, rank, src_base, phase, done, delta, s_acc, c_acc = st

    vs = []
    for k in range(4):
        vk = buf[pl.ds(src_base + 16 * k, 16)]
        vs.append(
            jnp.where(iota + 16 * k < n, vk, jnp.float32(_NEG_INF))
        )
    alo, ahi = _merge16(lax.sort(vs[0]), lax.sort(vs[1]))
    blo, bhi = _merge16(lax.sort(vs[2]), lax.sort(vs[3]))
    w = _merge32(alo, ahi, blo, bhi)
    idx = 64 - rank
    k_sel = idx // 16
    lane = idx % 16
    dsmall = jnp.float32(_NEG_INF)
    for k in range(4):
        dsmall = jnp.where(k_sel == k, _lane(w[k], lane), dsmall)
    delta = jnp.where(done == 0, dsmall, delta)
    delta_v = jnp.broadcast_to(delta, (16,))
    sv_acc = jnp.zeros((16,), jnp.float32)
    cv_acc = jnp.zeros((16,), jnp.int32)
    live = done == 0
    for k in range(4):
        mv = jnp.logical_and(live, vs[k] > delta_v)
        sv_acc = sv_acc + jnp.where(mv, vs[k], jnp.float32(0.0))
        cv_acc = cv_acc + plsc.all_reduce_population_count(mv)
    s_acc = s_acc + jnp.sum(sv_acc)
    c_acc = c_acc + jnp.max(cv_acc)
    return delta, s_acc, c_acc


def _normalize_row(buf, row_base, delta, s_acc, c_acc):
    wsum = s_acc - delta * c_acc.astype(jnp.float32)
    denom_v = jnp.broadcast_to(wsum + jnp.float32(_EPS), (16,))
    scale_v = jnp.ones((16,), jnp.float32) / denom_v
    delta_v = jnp.broadcast_to(delta, (16,))

    @plsc.parallel_loop(0, _N, 16, unroll=8)
    def _p2(i):
        sl = pl.ds(row_base + i, 16)
        buf[sl] = jnp.maximum(buf[sl] - delta_v, 0.0) * scale_v


def _sc_body(x_hbm, out_hbm, buf):
    wid = lax.axis_index("s") * _NC + lax.axis_index("c")
    base = wid * (_RPW * _N)
    pltpu.sync_copy(x_hbm.at[pl.ds(base, _RPW * _N)], buf.at[pl.ds(0, _CB)])
    for r in range(_RPW):
        row_base = r * _N
        delta, s_acc, c_acc = _select_threshold(buf, row_base)
        _normalize_row(buf, row_base, delta, s_acc, c_acc)
    pltpu.sync_copy(buf.at[pl.ds(0, _CB)], out_hbm.at[pl.ds(base, _RPW * _N)])


@jax.jit
def kernel(attn_s):
    b, one, n = attn_s.shape
    x = attn_s.reshape(b * n)
    mesh = plsc.VectorSubcoreMesh(
        core_axis_name="c", subcore_axis_name="s",
        num_cores=_NC, num_subcores=_NS,
    )
    out = pl.kernel(
        _sc_body,
        out_type=jax.ShapeDtypeStruct((b * n,), jnp.float32),
        mesh=mesh,
        scratch_types=[
            pltpu.VMEM((_CB + 4 * _N,), jnp.float32),
        ],
        compiler_params=pltpu.CompilerParams(needs_layout_passes=False),
    )(x)
    return out.reshape(b, one, n)
